# baseline (speedup 1.0000x reference)
import functools

import jax
import jax.numpy as jnp
from jax.experimental import pallas as pl
from jax.experimental.pallas import tpu as pltpu


def _conv_bd_kernel(x_ref, w_ref, b_ref, o_ref, xpad_ref, xcol_ref, *,
                    H, W, K, M, C, bt, pad_lanes):
    HW = H * W
    p = K // 2
    f32 = jnp.float32

    zeros_halo = jnp.zeros((C, pad_lanes), xpad_ref.dtype)
    xpad_ref[:, 0:pad_lanes] = zeros_halo
    xpad_ref[:, pad_lanes + HW:2 * pad_lanes + HW] = zeros_halo

    col = jax.lax.broadcasted_iota(jnp.int32, (1, HW), 1) % W
    col_masks = []
    for dx in range(K):
        dxo = dx - p
        if dxo == 0:
            col_masks.append(None)
        else:
            col_masks.append((col + dxo >= 0) & (col + dxo < W))

    bias = b_ref[...]
    for b in range(bt):
        xpad_ref[:, pad_lanes:pad_lanes + HW] = x_ref[b].astype(xpad_ref.dtype)
        for dy in range(K):
            for dx in range(K):
                t = dy * K + dx
                s = (dy - p) * W + (dx - p)
                win = xpad_ref[:, pad_lanes + s:pad_lanes + s + HW]
                if col_masks[dx] is not None:
                    win = jnp.where(col_masks[dx], win,
                                    jnp.zeros((), win.dtype))
                xcol_ref[t * C:(t + 1) * C, :] = win
        acc = jax.lax.dot_general(
            w_ref[...], xcol_ref[...],
            (((1,), (0,)), ((), ())),
            preferred_element_type=f32)
        o_ref[b] = (acc + bias).astype(o_ref.dtype)


def _conv_basis(x, weight, bias, basis_size, kernel_size):
    K = kernel_size
    T, N, C, H, W = x.shape
    n_basis = weight.shape[0]
    p = K // 2
    group = C // basis_size
    HW = H * W
    B = T * N
    M = group * n_basis

    pad_lanes = 128 * ((p * W + p + 127) // 128)

    wt = jnp.transpose(weight, (2, 3, 0, 1)).reshape(K * K, n_basis,
                                                     basis_size)
    eye = jnp.eye(group, dtype=weight.dtype)
    w_bd = jnp.einsum('gh,tnc->tgnhc', eye, wt).reshape(K * K, M, C)
    w2 = jnp.transpose(w_bd, (1, 0, 2)).reshape(M, K * K * C).astype(
        jnp.bfloat16)
    b_bd = jnp.tile(bias, group).reshape(M, 1).astype(jnp.float32)

    xr = x.reshape(B, C, HW)

    bt = 4
    while B % bt != 0:
        bt //= 2

    kfn = functools.partial(_conv_bd_kernel, H=H, W=W, K=K, M=M, C=C,
                            bt=bt, pad_lanes=pad_lanes)

    out = pl.pallas_call(
        kfn,
        out_shape=jax.ShapeDtypeStruct((B, M, HW), x.dtype),
        grid=(B // bt,),
        in_specs=[
            pl.BlockSpec((bt, C, HW), lambda i: (i, 0, 0)),
            pl.BlockSpec((M, K * K * C), lambda i: (0, 0)),
            pl.BlockSpec((M, 1), lambda i: (0, 0)),
        ],
        out_specs=pl.BlockSpec((bt, M, HW), lambda i: (i, 0, 0)),
        scratch_shapes=[
            pltpu.VMEM((C, HW + 2 * pad_lanes), jnp.bfloat16),
            pltpu.VMEM((K * K * C, HW), jnp.bfloat16),
        ],
        compiler_params=pltpu.CompilerParams(
            dimension_semantics=("parallel",),
            vmem_limit_bytes=48 * 1024 * 1024,
        ),
    )(xr, w2, b_bd)

    return out.reshape(T, N, M, H, W)


def kernel(x, weight, bias):
    return _conv_basis(x, weight, bias, 4, 3)
